# baseline (speedup 1.0000x reference)
import functools

import jax
import jax.numpy as jnp
from jax import lax
from jax.experimental import pallas as pl
from jax.experimental.pallas import tpu as pltpu
from jax.experimental.pallas import tpu_sc as plsc

D = 768
N = 8192
NC, NS = 2, 16
NW = NC * NS
BPW = N // NW
C = 64
NCHUNK = BPW // C


def _sc_gather(idx, table, n_out):
    mesh = plsc.VectorSubcoreMesh(core_axis_name="c", subcore_axis_name="s")
    out_t = (jax.ShapeDtypeStruct((N, D), jnp.float32),) * n_out

    @functools.partial(
        pl.kernel,
        out_type=out_t,
        mesh=mesh,
        scratch_types=[
            pltpu.VMEM((NCHUNK, C), jnp.int32),
            pltpu.VMEM((C, D), jnp.float32),
            pltpu.VMEM((C, D), jnp.float32),
            pltpu.SemaphoreType.DMA,
            pltpu.SemaphoreType.DMA,
            pltpu.SemaphoreType.DMA,
            pltpu.SemaphoreType.DMA,
        ],
    )
    def k(idx_hbm, T, *rest):
        outs = rest[:n_out]
        idx_v, rows0, rows1, g0, g1, w0, w1 = rest[n_out:]
        wid = lax.axis_index("s") * NC + lax.axis_index("c")
        base = wid * BPW
        pltpu.sync_copy(idx_hbm.at[wid], idx_v)
        rows = (rows0, rows1)
        gsem = (g0, g1)
        wsem = (w0, w1)

        def start_gather(c):
            b = c % 2
            return pltpu.async_copy(T.at[idx_v.at[c]], rows[b], gsem[b])

        def start_writes(c):
            b = c % 2
            sl = pl.ds(base + c * C, C)
            return [pltpu.async_copy(rows[b], O.at[sl], wsem[b])
                    for O in outs]

        g = [None, None]
        w = [None, None]
        g[0] = start_gather(0)
        for c in range(NCHUNK):
            b = c % 2
            if c + 1 < NCHUNK:
                nb = (c + 1) % 2
                if w[nb] is not None:
                    for h in w[nb]:
                        h.wait()
                    w[nb] = None
                g[nb] = start_gather(c + 1)
            g[b].wait()
            w[b] = start_writes(c)
        for hs in w:
            if hs is not None:
                for h in hs:
                    h.wait()

    return k(idx, table)


def _tc_copy(x):
    blk = N // 16

    def body(x_ref, o_ref):
        o_ref[...] = x_ref[...]

    return pl.pallas_call(
        body,
        grid=(16,),
        in_specs=[pl.BlockSpec((blk, D), lambda i: (i, 0))],
        out_specs=pl.BlockSpec((blk, D), lambda i: (i, 0)),
        out_shape=jax.ShapeDtypeStruct((N, D), jnp.float32),
    )(x)


def kernel(inputs, table0, table1, table2):
    B, S = inputs.shape
    idx = inputs.reshape(NW, NCHUNK, C).astype(jnp.int32)
    (o0,) = _sc_gather(idx, table0, 1)
    (o1,) = _sc_gather(idx, table1, 1)
    o2, o3 = _sc_gather(idx, table2, 2)
    o5 = _tc_copy(o0)
    o4 = _tc_copy(o1)
    return tuple(o.reshape(B, S, D) for o in (o0, o1, o2, o3, o4, o5))


# speedup vs baseline: 1.0524x; 1.0524x over previous
import functools

import jax
import jax.numpy as jnp
from jax import lax
from jax.experimental import pallas as pl
from jax.experimental.pallas import tpu as pltpu
from jax.experimental.pallas import tpu_sc as plsc

D = 768
N = 8192
NC, NS = 2, 16
NW = NC * NS
BPW = N // NW
C = 64
NCHUNK = BPW // C


def _sc_gather(idx, tables, n_outs):
    mesh = plsc.VectorSubcoreMesh(core_axis_name="c", subcore_axis_name="s")
    total_outs = sum(n_outs)
    out_t = (jax.ShapeDtypeStruct((N, D), jnp.float32),) * total_outs
    nt = len(tables)

    @functools.partial(
        pl.kernel,
        out_type=out_t,
        mesh=mesh,
        scratch_types=[
            pltpu.VMEM((NCHUNK, C), jnp.int32),
            pltpu.VMEM((C, D), jnp.float32),
            pltpu.VMEM((C, D), jnp.float32),
            pltpu.SemaphoreType.DMA,
            pltpu.SemaphoreType.DMA,
            pltpu.SemaphoreType.DMA,
            pltpu.SemaphoreType.DMA,
        ],
    )
    def k(idx_hbm, *rest):
        Ts = rest[:nt]
        outs = rest[nt:nt + total_outs]
        idx_v, rows0, rows1, g0, g1, w0, w1 = rest[nt + total_outs:]
        groups = []
        off = 0
        for i in range(nt):
            groups.append((Ts[i], outs[off:off + n_outs[i]]))
            off += n_outs[i]
        wid = lax.axis_index("s") * NC + lax.axis_index("c")
        base = wid * BPW
        pltpu.sync_copy(idx_hbm.at[wid], idx_v)
        tasks = [(T, Os, c) for (T, Os) in groups for c in range(NCHUNK)]
        rows = (rows0, rows1)
        gsem = (g0, g1)
        wsem = (w0, w1)

        def start_gather(i):
            T, _, c = tasks[i]
            b = i % 2
            return pltpu.async_copy(T.at[idx_v.at[c]], rows[b], gsem[b])

        def start_writes(i):
            _, Os, c = tasks[i]
            b = i % 2
            sl = pl.ds(base + c * C, C)
            return [pltpu.async_copy(rows[b], O.at[sl], wsem[b]) for O in Os]

        n = len(tasks)
        g = [None, None]
        w = [None, None]
        g[0] = start_gather(0)
        for i in range(n):
            b = i % 2
            if i + 1 < n:
                nb = (i + 1) % 2
                if w[nb] is not None:
                    for h in w[nb]:
                        h.wait()
                    w[nb] = None
                g[nb] = start_gather(i + 1)
            g[b].wait()
            w[b] = start_writes(i)
        for hs in w:
            if hs is not None:
                for h in hs:
                    h.wait()

    return k(idx, *tables)


def _tc_copy(x):
    blk = N // 32

    def body(x_ref, o_ref):
        o_ref[...] = x_ref[...]

    return pl.pallas_call(
        body,
        grid=(32,),
        in_specs=[pl.BlockSpec((blk, D), lambda i: (i, 0))],
        out_specs=pl.BlockSpec((blk, D), lambda i: (i, 0)),
        out_shape=jax.ShapeDtypeStruct((N, D), jnp.float32),
    )(x)


def kernel(inputs, table0, table1, table2):
    B, S = inputs.shape
    idx = inputs.reshape(NW, NCHUNK, C).astype(jnp.int32)
    (o0,) = _sc_gather(idx, (table0,), (1,))
    o1, o4, o2, o3 = _sc_gather(idx, (table1, table2), (2, 2))
    o5 = _tc_copy(o0)
    return tuple(o.reshape(B, S, D) for o in (o0, o1, o2, o3, o4, o5))


# speedup vs baseline: 1.1856x; 1.1266x over previous
import functools

import jax
import jax.numpy as jnp
from jax import lax
from jax.experimental import pallas as pl
from jax.experimental.pallas import tpu as pltpu
from jax.experimental.pallas import tpu_sc as plsc

D = 768
S_ = 2048
N = 8192
NC, NS = 2, 16
NW = NC * NS
BPW = N // NW
C = 64
NCHUNK = BPW // C


def _gather6(idx, t0, t1, t2):
    mesh = plsc.VectorSubcoreMesh(core_axis_name="c", subcore_axis_name="s")
    out_t = (jax.ShapeDtypeStruct((N, D), jnp.float32),) * 6

    @functools.partial(
        pl.kernel,
        out_type=out_t,
        mesh=mesh,
        scratch_types=[
            pltpu.VMEM((BPW,), jnp.int32),
            pltpu.VMEM((C, D), jnp.float32),
            pltpu.VMEM((C, D), jnp.float32),
            pltpu.SemaphoreType.DMA,
            pltpu.SemaphoreType.DMA,
            pltpu.SemaphoreType.DMA,
            pltpu.SemaphoreType.DMA,
        ],
    )
    def k(idx_hbm, T0, T1, T2, O0, O1, O2, O3, O4, O5, idx_v, rows0, rows1,
          g0, g1, w0, w1):
        wid = lax.axis_index("s") * NC + lax.axis_index("c")
        base = wid * BPW
        pltpu.sync_copy(
            idx_hbm.at[wid // (S_ // BPW), pl.ds((wid % (S_ // BPW)) * BPW, BPW)],
            idx_v)
        tasks = [(T, Oa, Ob, c)
                 for (T, Oa, Ob) in ((T0, O0, O5), (T1, O1, O4), (T2, O2, O3))
                 for c in range(NCHUNK)]
        rows = (rows0, rows1)
        gsem = (g0, g1)
        wsem = (w0, w1)

        def start_gather(i):
            T, _, _, c = tasks[i]
            b = i % 2
            return pltpu.async_copy(T.at[idx_v.at[pl.ds(c * C, C)]],
                                    rows[b], gsem[b])

        def start_writes(i):
            _, Oa, Ob, c = tasks[i]
            b = i % 2
            sl = pl.ds(base + c * C, C)
            ha = pltpu.async_copy(rows[b], Oa.at[sl], wsem[b])
            hb = pltpu.async_copy(rows[b], Ob.at[sl], wsem[b])
            return (ha, hb)

        n = len(tasks)
        g = [None, None]
        w = [None, None]
        g[0] = start_gather(0)
        for i in range(n):
            b = i % 2
            if i + 1 < n:
                nb = (i + 1) % 2
                if w[nb] is not None:
                    for h in w[nb]:
                        h.wait()
                    w[nb] = None
                g[nb] = start_gather(i + 1)
            g[b].wait()
            w[b] = start_writes(i)
        for pair in w:
            if pair is not None:
                for h in pair:
                    h.wait()

    return k(idx, t0, t1, t2)


def kernel(inputs, table0, table1, table2):
    B, S = inputs.shape
    outs = _gather6(inputs.astype(jnp.int32), table0, table1, table2)
    return tuple(o.reshape(B, S, D) for o in outs)
